# baseline (speedup 1.0000x reference)
import functools

import jax
import jax.numpy as jnp
from jax import lax
from jax.experimental import pallas as pl
from jax.experimental.pallas import tpu as pltpu
from jax.experimental.pallas import tpu_sc as plsc

N = 10000
E = 320000
D = 128
G = 64
L = 3

NC = 2
NS = 16
NW = NC * NS
NPAD = 10008
RPS = 632
RPSL = NPAD - RPS * (NS - 1)
WIN = 128
NI = 81
NWINP = NI * NW
EPAD = NWINP * WIN
_mesh = plsc.VectorSubcoreMesh(core_axis_name="c", subcore_axis_name="s")


@functools.partial(
    pl.kernel,
    out_type=jax.ShapeDtypeStruct((NC, NPAD, D), jnp.float32),
    mesh=_mesh,
    scratch_types=[
        pltpu.VMEM_SHARED((NPAD, D), jnp.float32),
        [pltpu.VMEM((WIN,), jnp.int32)] * 3,
        [pltpu.VMEM((WIN,), jnp.int32)] * 3,
        [pltpu.VMEM((WIN, D), jnp.float32)] * 3,
        pltpu.SemaphoreType.DMA,
        pltpu.SemaphoreType.DMA,
        pltpu.SemaphoreType.DMA,
    ],
)
def _sc_agg_kernel(h_hbm, e_hbm, z_hbm, out_hbm, acc, sb, db, rb, sem, semi, semz):
    c = lax.axis_index("c")
    s = lax.axis_index("s")
    wid = s * NC + c

    zd = [None]

    @pl.when(s < NS - 1)
    def _():
        zd[0] = pltpu.async_copy(z_hbm, acc.at[pl.ds(s * RPS, RPS)], semz)

    @pl.when(s == NS - 1)
    def _():
        zd[0] = pltpu.async_copy(z_hbm.at[pl.ds(0, RPSL)],
                                 acc.at[pl.ds((NS - 1) * RPS, RPSL)], semz)

    def idx_fire(v, sl):
        off = (wid + NW * v) * WIN
        return (pltpu.async_copy(e_hbm.at[0, pl.ds(off, WIN)], sb[sl], semi),
                pltpu.async_copy(e_hbm.at[1, pl.ds(off, WIN)], db[sl], semi))

    def idx_wait(sl):
        pltpu.make_async_copy(e_hbm.at[0, pl.ds(0, WIN)], sb[sl], semi).wait()
        pltpu.make_async_copy(e_hbm.at[1, pl.ds(0, WIN)], db[sl], semi).wait()

    def g_fire(sl):
        return pltpu.async_copy(h_hbm.at[sb[sl]], rb[sl], sem)

    def scat(sl):
        pltpu.sync_copy(rb[sl], acc.at[db[sl]], add=True)

    for d in (*idx_fire(0, 0), *idx_fire(1, 1)):
        d.wait()
    pltpu.sync_copy(h_hbm.at[sb[0]], rb[0])
    g_fire(1)
    idx_fire(2, 2)

    zd[0].wait()
    plsc.subcore_barrier()

    @pl.loop(0, 78, step=3)
    def _(vb):
        for u in range(3):
            v = vb + u
            sl2 = (u + 2) % 3
            idx_wait(sl2)
            g_fire(sl2)
            scat(u % 3)
            idx_fire(v + 3, u % 3)
            pltpu.make_async_copy(h_hbm.at[sb[(u + 1) % 3]],
                                  rb[(u + 1) % 3], sem).wait()

    idx_wait(2)
    g80 = g_fire(2)
    scat(0)
    pltpu.make_async_copy(h_hbm.at[sb[1]], rb[1], sem).wait()
    scat(1)
    g80.wait()
    scat(2)

    plsc.subcore_barrier()

    @pl.when(s < NS - 1)
    def _():
        pltpu.sync_copy(acc.at[pl.ds(s * RPS, RPS)],
                        out_hbm.at[c, pl.ds(s * RPS, RPS)])

    @pl.when(s == NS - 1)
    def _():
        pltpu.sync_copy(acc.at[pl.ds((NS - 1) * RPS, RPSL)],
                        out_hbm.at[c, pl.ds((NS - 1) * RPS, RPSL)])


def _sc_agg(h, edge_index3, zeros):
    return _sc_agg_kernel(h, edge_index3, zeros)


def _dot_t(a, w):
    return lax.dot_general(a, w, (((1,), (1,)), ((), ())),
                           preferred_element_type=jnp.float32)


def _tc_init_body(x_ref, w_ref, b_ref, o_ref):
    o_ref[...] = _dot_t(x_ref[...], w_ref[...]) + b_ref[...]


def _tc_init(x, W_init, b2):
    return pl.pallas_call(
        _tc_init_body,
        out_shape=jax.ShapeDtypeStruct((N, D), jnp.float32),
    )(x, W_init, b2)


def _tc_layer_body(h_ref, p_ref, wr_ref, br_ref, wt_ref, g_ref, b_ref, o_ref):
    agg = p_ref[0, :N, :] + p_ref[1, :N, :]
    h = h_ref[...]
    t = h + _dot_t(agg, wr_ref[...]) + br_ref[...] + _dot_t(h, wt_ref[...])
    m = jnp.mean(t, axis=0, keepdims=True)
    v = jnp.mean((t - m) ** 2, axis=0, keepdims=True)
    t = (t - m) / jnp.sqrt(v + 1e-5) * g_ref[...] + b_ref[...]
    o_ref[...] = jnp.maximum(t, 0.0)


def _tc_layer(h, parts, Wr, br2, Wt, g2, b2):
    return pl.pallas_call(
        _tc_layer_body,
        out_shape=jax.ShapeDtypeStruct((N, D), jnp.float32),
    )(h, parts, Wr, br2, Wt, g2, b2)


def _tc_final_body(h_ref, p_ref, wr_ref, br_ref, wt_ref, batch_ref, o_ref):
    agg = p_ref[0, :N, :] + p_ref[1, :N, :]
    t = _dot_t(agg, wr_ref[...]) + br_ref[...] + _dot_t(h_ref[...], wt_ref[...])
    seg = lax.broadcasted_iota(jnp.int32, (G, N), 0)
    mask = (seg == batch_ref[...]).astype(jnp.float32)
    o_ref[...] = lax.dot_general(mask, t, (((1,), (0,)), ((), ())),
                                 preferred_element_type=jnp.float32)


def _tc_final(h, parts, Wr, br2, Wt, batch2):
    return pl.pallas_call(
        _tc_final_body,
        out_shape=jax.ShapeDtypeStruct((G, D), jnp.float32),
    )(h, parts, Wr, br2, Wt, batch2)


def kernel(x, edge_index, batch, W_init, b_init, W_rel, b_rel, W_root, gamma, beta):
    zeros = jnp.zeros((RPS, D), jnp.float32)
    batch2 = batch.reshape(1, N)
    pad = jnp.stack([jnp.zeros((EPAD - E,), jnp.int32),
                     jnp.full((EPAD - E,), N, jnp.int32)])
    e3 = jnp.concatenate([edge_index, pad], axis=1)
    h = _tc_init(x, W_init, b_init.reshape(1, D))
    for i in range(L - 1):
        parts = _sc_agg(h, e3, zeros)
        h = _tc_layer(h, parts, W_rel[i], b_rel[i].reshape(1, D),
                      W_root[i], gamma[i].reshape(1, D), beta[i].reshape(1, D))
    parts = _sc_agg(h, e3, zeros)
    return _tc_final(h, parts, W_rel[L - 1], b_rel[L - 1].reshape(1, D),
                     W_root[L - 1], batch2)


# speedup vs baseline: 3.1001x; 3.1001x over previous
import functools

import jax
import jax.numpy as jnp
from jax import lax
from jax.experimental import pallas as pl
from jax.experimental.pallas import tpu as pltpu
from jax.experimental.pallas import tpu_sc as plsc

N = 10000
E = 320000
D = 128
G = 64
L = 3

NC = 2
NS = 16
NW = NC * NS
NPAD = 10008
RPS = 632
RPSL = NPAD - RPS * (NS - 1)
WIN = 128
NI = 81
NWINP = NI * NW
EPAD = NWINP * WIN
_mesh = plsc.VectorSubcoreMesh(core_axis_name="c", subcore_axis_name="s")


@functools.partial(
    pl.kernel,
    out_type=jax.ShapeDtypeStruct((NC, NPAD, D), jnp.float32),
    mesh=_mesh,
    scratch_types=[
        pltpu.VMEM_SHARED((NPAD, D), jnp.float32),
        [pltpu.VMEM((WIN,), jnp.int32)] * 3,
        [pltpu.VMEM((WIN,), jnp.int32)] * 3,
        [pltpu.VMEM((WIN, D), jnp.float32)] * 3,
        pltpu.SemaphoreType.DMA,
        pltpu.SemaphoreType.DMA,
        pltpu.SemaphoreType.DMA,
    ],
)
def _sc_agg_kernel(h_hbm, e_hbm, z_hbm, out_hbm, acc, sb, db, rb, sem, semi, semz):
    c = lax.axis_index("c")
    s = lax.axis_index("s")
    wid = s * NC + c

    zd = [None]

    @pl.when(s < NS - 1)
    def _():
        zd[0] = pltpu.async_copy(z_hbm, acc.at[pl.ds(s * RPS, RPS)], semz)

    @pl.when(s == NS - 1)
    def _():
        zd[0] = pltpu.async_copy(z_hbm.at[pl.ds(0, RPSL)],
                                 acc.at[pl.ds((NS - 1) * RPS, RPSL)], semz)

    def idx_fire(v, sl):
        off = (wid + NW * v) * WIN
        return (pltpu.async_copy(e_hbm.at[0, pl.ds(off, WIN)], sb[sl], semi),
                pltpu.async_copy(e_hbm.at[1, pl.ds(off, WIN)], db[sl], semi))

    def idx_wait(sl):
        pltpu.make_async_copy(e_hbm.at[0, pl.ds(0, WIN)], sb[sl], semi).wait()
        pltpu.make_async_copy(e_hbm.at[1, pl.ds(0, WIN)], db[sl], semi).wait()

    def g_fire(sl):
        return pltpu.async_copy(h_hbm.at[sb[sl]], rb[sl], sem)

    def scat(sl):
        pltpu.sync_copy(rb[sl], acc.at[db[sl]], add=True)

    for d in (*idx_fire(0, 0), *idx_fire(1, 1)):
        d.wait()
    pltpu.sync_copy(h_hbm.at[sb[0]], rb[0])
    g_fire(1)
    idx_fire(2, 2)

    zd[0].wait()
    plsc.subcore_barrier()

    @pl.loop(0, 78, step=3)
    def _(vb):
        for u in range(3):
            v = vb + u
            sl2 = (u + 2) % 3
            idx_wait(sl2)
            g_fire(sl2)
            scat(u % 3)
            idx_fire(v + 3, u % 3)
            pltpu.make_async_copy(h_hbm.at[sb[(u + 1) % 3]],
                                  rb[(u + 1) % 3], sem).wait()

    idx_wait(2)
    g80 = g_fire(2)
    scat(0)
    pltpu.make_async_copy(h_hbm.at[sb[1]], rb[1], sem).wait()
    scat(1)
    g80.wait()
    scat(2)

    plsc.subcore_barrier()

    @pl.when(s < NS - 1)
    def _():
        pltpu.sync_copy(acc.at[pl.ds(s * RPS, RPS)],
                        out_hbm.at[c, pl.ds(s * RPS, RPS)])

    @pl.when(s == NS - 1)
    def _():
        pltpu.sync_copy(acc.at[pl.ds((NS - 1) * RPS, RPSL)],
                        out_hbm.at[c, pl.ds((NS - 1) * RPS, RPSL)])


def _sc_agg(h, edge_index3, zeros):
    return _sc_agg_kernel(h, edge_index3, zeros)


def _dot_t(a, w):
    return lax.dot_general(a, w, (((1,), (1,)), ((), ())),
                           preferred_element_type=jnp.float32)


def _tc_init_body(x_ref, w_ref, b_ref, o_ref):
    o_ref[:N, :] = _dot_t(x_ref[...], w_ref[...]) + b_ref[...]
    o_ref[N:, :] = jnp.zeros((NPAD - N, D), jnp.float32)


def _tc_init(x, W_init, b2):
    return pl.pallas_call(
        _tc_init_body,
        out_shape=jax.ShapeDtypeStruct((NPAD, D), jnp.float32),
    )(x, W_init, b2)


def _tc_layer_body(h_ref, p_ref, wr_ref, br_ref, wt_ref, g_ref, b_ref, o_ref):
    agg = p_ref[0, :N, :] + p_ref[1, :N, :]
    h = h_ref[:N, :]
    t = h + _dot_t(agg, wr_ref[...]) + br_ref[...] + _dot_t(h, wt_ref[...])
    m = jnp.mean(t, axis=0, keepdims=True)
    v = jnp.mean((t - m) ** 2, axis=0, keepdims=True)
    t = (t - m) / jnp.sqrt(v + 1e-5) * g_ref[...] + b_ref[...]
    o_ref[:N, :] = jnp.maximum(t, 0.0)
    o_ref[N:, :] = jnp.zeros((NPAD - N, D), jnp.float32)


def _tc_layer(h, parts, Wr, br2, Wt, g2, b2):
    return pl.pallas_call(
        _tc_layer_body,
        out_shape=jax.ShapeDtypeStruct((NPAD, D), jnp.float32),
    )(h, parts, Wr, br2, Wt, g2, b2)


def _tc_final_body(h_ref, p_ref, wr_ref, br_ref, wt_ref, batch_ref, o_ref):
    agg = p_ref[0, :N, :] + p_ref[1, :N, :]
    t = _dot_t(agg, wr_ref[...]) + br_ref[...] + _dot_t(h_ref[:N, :], wt_ref[...])
    seg = lax.broadcasted_iota(jnp.int32, (G, N), 0)
    mask = (seg == batch_ref[...]).astype(jnp.float32)
    o_ref[...] = lax.dot_general(mask, t, (((1,), (0,)), ((), ())),
                                 preferred_element_type=jnp.float32)


def _tc_final(h, parts, Wr, br2, Wt, batch2):
    return pl.pallas_call(
        _tc_final_body,
        out_shape=jax.ShapeDtypeStruct((G, D), jnp.float32),
    )(h, parts, Wr, br2, Wt, batch2)


def kernel(x, edge_index, batch, W_init, b_init, W_rel, b_rel, W_root, gamma, beta):
    zeros = jnp.zeros((RPS, D), jnp.float32)
    batch2 = batch.reshape(1, N)
    j = jnp.arange(EPAD - E, dtype=jnp.int32)
    pad = jnp.stack([N + (j % (NPAD - N)), j % N])
    e3 = jnp.concatenate([edge_index, pad], axis=1)
    h = _tc_init(x, W_init, b_init.reshape(1, D))
    for i in range(L - 1):
        parts = _sc_agg(h, e3, zeros)
        h = _tc_layer(h, parts, W_rel[i], b_rel[i].reshape(1, D),
                      W_root[i], gamma[i].reshape(1, D), beta[i].reshape(1, D))
    parts = _sc_agg(h, e3, zeros)
    return _tc_final(h, parts, W_rel[L - 1], b_rel[L - 1].reshape(1, D),
                     W_root[L - 1], batch2)


# speedup vs baseline: 3.4509x; 1.1131x over previous
import functools

import jax
import jax.numpy as jnp
from jax import lax
from jax.experimental import pallas as pl
from jax.experimental.pallas import tpu as pltpu
from jax.experimental.pallas import tpu_sc as plsc

N = 10000
E = 320000
D = 128
G = 64
L = 3

NC = 2
NS = 16
NW = NC * NS
NPAD = 10008
RPS = 632
RPSL = NPAD - RPS * (NS - 1)
WIN = 128
NI = 81
NWINP = NI * NW
EPAD = NWINP * WIN
_mesh = plsc.VectorSubcoreMesh(core_axis_name="c", subcore_axis_name="s")


@functools.partial(
    pl.kernel,
    out_type=jax.ShapeDtypeStruct((NC, NPAD, D), jnp.float32),
    mesh=_mesh,
    scratch_types=[
        pltpu.VMEM_SHARED((NPAD, D), jnp.float32),
        [pltpu.VMEM((WIN,), jnp.int32)] * 4,
        [pltpu.VMEM((WIN,), jnp.int32)] * 4,
        [pltpu.VMEM((WIN, D), jnp.float32)] * 3,
        pltpu.SemaphoreType.DMA,
        pltpu.SemaphoreType.DMA,
        pltpu.SemaphoreType.DMA,
    ],
)
def _sc_agg_kernel(h_hbm, e_hbm, z_hbm, out_hbm, acc, sb, db, rb, sem, semi, semz):
    c = lax.axis_index("c")
    s = lax.axis_index("s")
    wid = s * NC + c

    zd = [None]

    @pl.when(s < NS - 1)
    def _():
        zd[0] = pltpu.async_copy(z_hbm, acc.at[pl.ds(s * RPS, RPS)], semz)

    @pl.when(s == NS - 1)
    def _():
        zd[0] = pltpu.async_copy(z_hbm.at[pl.ds(0, RPSL)],
                                 acc.at[pl.ds((NS - 1) * RPS, RPSL)], semz)

    def idx_fire(v, sl):
        off = (wid + NW * v) * WIN
        return (pltpu.async_copy(e_hbm.at[0, pl.ds(off, WIN)], sb[sl], semi),
                pltpu.async_copy(e_hbm.at[1, pl.ds(off, WIN)], db[sl], semi))

    def idx_wait(sl):
        pltpu.make_async_copy(e_hbm.at[0, pl.ds(0, WIN)], sb[sl], semi).wait()
        pltpu.make_async_copy(e_hbm.at[1, pl.ds(0, WIN)], db[sl], semi).wait()

    def g_fire(sl, rsl):
        return pltpu.async_copy(h_hbm.at[sb[sl]], rb[rsl], sem)

    def scat(sl, rsl):
        pltpu.sync_copy(rb[rsl], acc.at[db[sl]], add=True)

    for d in (*idx_fire(0, 0), *idx_fire(1, 1)):
        d.wait()
    pltpu.sync_copy(h_hbm.at[sb[0]], rb[0])
    g_fire(1, 1)
    idx_fire(2, 2)

    zd[0].wait()
    plsc.subcore_barrier()

    @pl.loop(0, 72, step=12)
    def _(vb):
        for u in range(12):
            v = vb + u
            idx_wait((u + 2) & 3)
            g_fire((u + 2) & 3, (u + 2) % 3)
            idx_fire(v + 3, (u + 3) & 3)
            scat((u + 0) & 3, (u + 0) % 3)
            pltpu.make_async_copy(h_hbm.at[sb[(u + 1) & 3]],
                                  rb[(u + 1) % 3], sem).wait()

    for v in range(72, 81):
        u = v - 72
        if v + 2 <= 80:
            idx_wait((u + 2) & 3)
            g_fire((u + 2) & 3, (u + 2) % 3)
        if v + 3 <= 80:
            idx_fire(v + 3, (u + 3) & 3)
        scat(u & 3, u % 3)
        if v + 1 <= 80:
            pltpu.make_async_copy(h_hbm.at[sb[(u + 1) & 3]],
                                  rb[(u + 1) % 3], sem).wait()

    plsc.subcore_barrier()

    @pl.when(s < NS - 1)
    def _():
        pltpu.sync_copy(acc.at[pl.ds(s * RPS, RPS)],
                        out_hbm.at[c, pl.ds(s * RPS, RPS)])

    @pl.when(s == NS - 1)
    def _():
        pltpu.sync_copy(acc.at[pl.ds((NS - 1) * RPS, RPSL)],
                        out_hbm.at[c, pl.ds((NS - 1) * RPS, RPSL)])


def _sc_agg(h, edge_index3, zeros):
    return _sc_agg_kernel(h, edge_index3, zeros)


def _dot_t(a, w):
    return lax.dot_general(a, w, (((1,), (1,)), ((), ())),
                           preferred_element_type=jnp.float32)


def _tc_init_body(x_ref, w_ref, b_ref, o_ref):
    o_ref[:N, :] = _dot_t(x_ref[...], w_ref[...]) + b_ref[...]
    o_ref[N:, :] = jnp.zeros((NPAD - N, D), jnp.float32)


def _tc_init(x, W_init, b2):
    return pl.pallas_call(
        _tc_init_body,
        out_shape=jax.ShapeDtypeStruct((NPAD, D), jnp.float32),
    )(x, W_init, b2)


def _tc_layer_body(h_ref, p_ref, wr_ref, br_ref, wt_ref, g_ref, b_ref, o_ref):
    agg = p_ref[0, :N, :] + p_ref[1, :N, :]
    h = h_ref[:N, :]
    t = h + _dot_t(agg, wr_ref[...]) + br_ref[...] + _dot_t(h, wt_ref[...])
    m = jnp.mean(t, axis=0, keepdims=True)
    v = jnp.mean((t - m) ** 2, axis=0, keepdims=True)
    t = (t - m) / jnp.sqrt(v + 1e-5) * g_ref[...] + b_ref[...]
    o_ref[:N, :] = jnp.maximum(t, 0.0)
    o_ref[N:, :] = jnp.zeros((NPAD - N, D), jnp.float32)


def _tc_layer(h, parts, Wr, br2, Wt, g2, b2):
    return pl.pallas_call(
        _tc_layer_body,
        out_shape=jax.ShapeDtypeStruct((NPAD, D), jnp.float32),
    )(h, parts, Wr, br2, Wt, g2, b2)


def _tc_final_body(h_ref, p_ref, wr_ref, br_ref, wt_ref, batch_ref, o_ref):
    agg = p_ref[0, :N, :] + p_ref[1, :N, :]
    t = _dot_t(agg, wr_ref[...]) + br_ref[...] + _dot_t(h_ref[:N, :], wt_ref[...])
    seg = lax.broadcasted_iota(jnp.int32, (G, N), 0)
    mask = (seg == batch_ref[...]).astype(jnp.float32)
    o_ref[...] = lax.dot_general(mask, t, (((1,), (0,)), ((), ())),
                                 preferred_element_type=jnp.float32)


def _tc_final(h, parts, Wr, br2, Wt, batch2):
    return pl.pallas_call(
        _tc_final_body,
        out_shape=jax.ShapeDtypeStruct((G, D), jnp.float32),
    )(h, parts, Wr, br2, Wt, batch2)


def kernel(x, edge_index, batch, W_init, b_init, W_rel, b_rel, W_root, gamma, beta):
    zeros = jnp.zeros((RPS, D), jnp.float32)
    batch2 = batch.reshape(1, N)
    j = jnp.arange(EPAD - E, dtype=jnp.int32)
    pad = jnp.stack([N + (j % (NPAD - N)), j % N])
    e3 = jnp.concatenate([edge_index, pad], axis=1)
    h = _tc_init(x, W_init, b_init.reshape(1, D))
    for i in range(L - 1):
        parts = _sc_agg(h, e3, zeros)
        h = _tc_layer(h, parts, W_rel[i], b_rel[i].reshape(1, D),
                      W_root[i], gamma[i].reshape(1, D), beta[i].reshape(1, D))
    parts = _sc_agg(h, e3, zeros)
    return _tc_final(h, parts, W_rel[L - 1], b_rel[L - 1].reshape(1, D),
                     W_root[L - 1], batch2)


# speedup vs baseline: 4.5152x; 1.3084x over previous
import functools

import jax
import jax.numpy as jnp
from jax import lax
from jax.experimental import pallas as pl
from jax.experimental.pallas import tpu as pltpu
from jax.experimental.pallas import tpu_sc as plsc

N = 10000
E = 320000
D = 128
G = 64
L = 3

NC = 2
NS = 16
NW = NC * NS
NPAD = 10008
RPS = 632
RPSL = NPAD - RPS * (NS - 1)
WIN = 128
NI = 79
NWINP = NI * NW
EPAD = NWINP * WIN
_mesh = plsc.VectorSubcoreMesh(core_axis_name="c", subcore_axis_name="s")


@functools.partial(
    pl.kernel,
    out_type=jax.ShapeDtypeStruct((NC, NPAD, D), jnp.float32),
    mesh=_mesh,
    scratch_types=[
        pltpu.VMEM_SHARED((NPAD, D), jnp.float32),
        [pltpu.VMEM((WIN,), jnp.int32)] * 4,
        [pltpu.VMEM((WIN,), jnp.int32)] * 4,
        [pltpu.VMEM((WIN, D), jnp.float32)] * 3,
        pltpu.SemaphoreType.DMA,
        pltpu.SemaphoreType.DMA,
        pltpu.SemaphoreType.DMA,
    ],
)
def _sc_agg_kernel(h_hbm, e_hbm, z_hbm, out_hbm, acc, sb, db, rb, sem, semi, semz):
    c = lax.axis_index("c")
    s = lax.axis_index("s")
    wid = s * NC + c

    zd = [None]

    @pl.when(s < NS - 1)
    def _():
        zd[0] = pltpu.async_copy(z_hbm, acc.at[pl.ds(s * RPS, RPS)], semz)

    @pl.when(s == NS - 1)
    def _():
        zd[0] = pltpu.async_copy(z_hbm.at[pl.ds(0, RPSL)],
                                 acc.at[pl.ds((NS - 1) * RPS, RPSL)], semz)

    def idx_fire(v, sl):
        off = (wid + NW * v) * WIN
        return (pltpu.async_copy(e_hbm.at[0, pl.ds(off, WIN)], sb[sl], semi),
                pltpu.async_copy(e_hbm.at[1, pl.ds(off, WIN)], db[sl], semi))

    def idx_wait(sl):
        pltpu.make_async_copy(e_hbm.at[0, pl.ds(0, WIN)], sb[sl], semi).wait()
        pltpu.make_async_copy(e_hbm.at[1, pl.ds(0, WIN)], db[sl], semi).wait()

    def g_fire(sl, rsl):
        return pltpu.async_copy(h_hbm.at[sb[sl]], rb[rsl], sem)

    def scat(sl, rsl):
        pltpu.sync_copy(rb[rsl], acc.at[db[sl]], add=True)

    for d in (*idx_fire(0, 0), *idx_fire(1, 1)):
        d.wait()
    pltpu.sync_copy(h_hbm.at[sb[0]], rb[0])
    g_fire(1, 1)
    idx_fire(2, 2)

    zd[0].wait()
    plsc.subcore_barrier()

    @pl.loop(0, 72, step=12)
    def _(vb):
        for u in range(12):
            v = vb + u
            idx_wait((u + 2) & 3)
            g_fire((u + 2) & 3, (u + 2) % 3)
            idx_fire(v + 3, (u + 3) & 3)
            scat((u + 0) & 3, (u + 0) % 3)
            pltpu.make_async_copy(h_hbm.at[sb[(u + 1) & 3]],
                                  rb[(u + 1) % 3], sem).wait()

    for v in range(72, NI):
        u = v - 72
        if v + 2 <= NI - 1:
            idx_wait((u + 2) & 3)
            g_fire((u + 2) & 3, (u + 2) % 3)
        if v + 3 <= NI - 1:
            idx_fire(v + 3, (u + 3) & 3)
        scat(u & 3, u % 3)
        if v + 1 <= NI - 1:
            pltpu.make_async_copy(h_hbm.at[sb[(u + 1) & 3]],
                                  rb[(u + 1) % 3], sem).wait()

    plsc.subcore_barrier()

    @pl.when(s < NS - 1)
    def _():
        pltpu.sync_copy(acc.at[pl.ds(s * RPS, RPS)],
                        out_hbm.at[c, pl.ds(s * RPS, RPS)])

    @pl.when(s == NS - 1)
    def _():
        pltpu.sync_copy(acc.at[pl.ds((NS - 1) * RPS, RPSL)],
                        out_hbm.at[c, pl.ds((NS - 1) * RPS, RPSL)])


def _sc_agg(h, edge_index3, zeros):
    return _sc_agg_kernel(h, edge_index3, zeros)


def _dot_t(a, w):
    return lax.dot_general(a, w, (((1,), (1,)), ((), ())),
                           preferred_element_type=jnp.float32)


def _tc_init_body(x_ref, w_ref, b_ref, o_ref):
    o_ref[:N, :] = _dot_t(x_ref[...], w_ref[...]) + b_ref[...]
    o_ref[N:, :] = jnp.zeros((NPAD - N, D), jnp.float32)


def _tc_init(x, W_init, b2):
    return pl.pallas_call(
        _tc_init_body,
        out_shape=jax.ShapeDtypeStruct((NPAD, D), jnp.float32),
    )(x, W_init, b2)


def _tc_layer_body(h_ref, p_ref, wr_ref, br_ref, wt_ref, g_ref, b_ref, o_ref):
    agg = p_ref[0, :N, :] + p_ref[1, :N, :]
    h = h_ref[:N, :]
    t = h + _dot_t(agg, wr_ref[...]) + br_ref[...] + _dot_t(h, wt_ref[...])
    m = jnp.mean(t, axis=0, keepdims=True)
    v = jnp.mean((t - m) ** 2, axis=0, keepdims=True)
    t = (t - m) / jnp.sqrt(v + 1e-5) * g_ref[...] + b_ref[...]
    o_ref[:N, :] = jnp.maximum(t, 0.0)
    o_ref[N:, :] = jnp.zeros((NPAD - N, D), jnp.float32)


def _tc_layer(h, parts, Wr, br2, Wt, g2, b2):
    return pl.pallas_call(
        _tc_layer_body,
        out_shape=jax.ShapeDtypeStruct((NPAD, D), jnp.float32),
    )(h, parts, Wr, br2, Wt, g2, b2)


def _tc_final_body(h_ref, p_ref, wr_ref, br_ref, wt_ref, batch_ref, o_ref):
    agg = p_ref[0, :N, :] + p_ref[1, :N, :]
    t = _dot_t(agg, wr_ref[...]) + br_ref[...] + _dot_t(h_ref[:N, :], wt_ref[...])
    seg = lax.broadcasted_iota(jnp.int32, (G, N), 0)
    mask = (seg == batch_ref[...]).astype(jnp.float32)
    o_ref[...] = lax.dot_general(mask, t, (((1,), (0,)), ((), ())),
                                 preferred_element_type=jnp.float32)


def _tc_final(h, parts, Wr, br2, Wt, batch2):
    return pl.pallas_call(
        _tc_final_body,
        out_shape=jax.ShapeDtypeStruct((G, D), jnp.float32),
    )(h, parts, Wr, br2, Wt, batch2)


def kernel(x, edge_index, batch, W_init, b_init, W_rel, b_rel, W_root, gamma, beta):
    zeros = jnp.zeros((RPS, D), jnp.float32)
    batch2 = batch.reshape(1, N)
    j = jnp.arange(EPAD - E, dtype=jnp.int32)
    pad = jnp.stack([N + (j % (NPAD - N)), j % N])
    e3 = jnp.concatenate([edge_index, pad], axis=1)
    h = _tc_init(x, W_init, b_init.reshape(1, D))
    for i in range(L - 1):
        parts = _sc_agg(h, e3, zeros)
        h = _tc_layer(h, parts, W_rel[i], b_rel[i].reshape(1, D),
                      W_root[i], gamma[i].reshape(1, D), beta[i].reshape(1, D))
    parts = _sc_agg(h, e3, zeros)
    return _tc_final(h, parts, W_rel[L - 1], b_rel[L - 1].reshape(1, D),
                     W_root[L - 1], batch2)
